# Initial kernel scaffold; baseline (speedup 1.0000x reference)
import jax
import jax.numpy as jnp
from jax.experimental import pallas as pl


def kernel(x, nbrs_idx, t, e_hat, W1, b1, W2, b2, W3, b3, b):
    raise NotImplementedError("write your pallas kernel here")


# speedup vs baseline: 8.3728x; 8.3728x over previous
import functools

import jax
import jax.numpy as jnp
from jax import lax
from jax.experimental import pallas as pl
from jax.experimental.pallas import tpu as pltpu
from jax.experimental.pallas import tpu_sc as plsc

N = 4096
D = 128
H = 128
K = 32
E = N * K

NC = 2
NS = 16
NW = NC * NS

EPW = E // NW
GJ = EPW // 128

_sc_mesh = plsc.VectorSubcoreMesh(core_axis_name="c", subcore_axis_name="s")


@functools.partial(
    pl.kernel,
    out_type=(
        jax.ShapeDtypeStruct((E, D), jnp.float32),
        jax.ShapeDtypeStruct((E,), jnp.float32),
        jax.ShapeDtypeStruct((E,), jnp.float32),
    ),
    mesh=_sc_mesh,
    compiler_params=pltpu.CompilerParams(needs_layout_passes=False),
    scratch_types=[
        pltpu.VMEM((EPW,), jnp.int32),
        pltpu.VMEM((2, 128, D), jnp.float32),
        pltpu.VMEM((EPW,), jnp.float32),
        pltpu.VMEM((EPW,), jnp.float32),
        pltpu.SemaphoreType.DMA,
        pltpu.SemaphoreType.DMA,
        pltpu.SemaphoreType.DMA,
    ],
)
def _gather_sc(x_hbm, nbrs_hbm, t_hbm, e_hbm, z_out, tg_out, eg_out,
               idx_v, ring_v, t_v, e_v, gsem, osem, tsem):
    wid = lax.axis_index("s") * NC + lax.axis_index("c")
    base = wid * EPW

    pltpu.sync_copy(nbrs_hbm.at[pl.ds(base, EPW)], idx_v)

    def _fire(j, b):
        idxref = idx_v.at[pl.ds(j * 128, 128)]
        dt = pltpu.async_copy(t_hbm.at[idxref], t_v.at[pl.ds(j * 128, 128)], tsem)
        de = pltpu.async_copy(e_hbm.at[idxref], e_v.at[pl.ds(j * 128, 128)], tsem)
        dz = pltpu.async_copy(x_hbm.at[idxref], ring_v.at[b], gsem)
        return (dz, dt, de)

    out_descs = [None, None]
    g_desc = _fire(0, 0)
    for j in range(GJ):
        b = j % 2
        nb = (j + 1) % 2
        g_cur = g_desc
        if j + 1 < GJ:
            if out_descs[nb] is not None:
                out_descs[nb].wait()
                out_descs[nb] = None
            g_desc = _fire(j + 1, nb)
        for d in g_cur:
            d.wait()
        out_descs[b] = pltpu.async_copy(
            ring_v.at[b], z_out.at[pl.ds(base + j * 128, 128)], osem)
    for od in out_descs:
        if od is not None:
            od.wait()
    pltpu.sync_copy(t_v, tg_out.at[pl.ds(base, EPW)])
    pltpu.sync_copy(e_v, eg_out.at[pl.ds(base, EPW)])


RB = 128
GRID = N // RB


def _mlp_body(z_ref, nbrs_ref, tg_ref, eg_ref, w1_ref, b1_ref, w2_ref, b2_ref,
              w3_ref, b3_ref, bs_ref, pw_ref, y_ref, addr_ref):
    z = z_ref[...]
    w1 = w1_ref[...]
    zi = z.reshape(RB, K, D)[:, 0, :]
    vi = jnp.dot(zi, w1[:D, :], preferred_element_type=jnp.float32)
    u = jnp.dot(z, w1[D:, :], preferred_element_type=jnp.float32)
    h1 = jnp.maximum(
        u.reshape(RB, K, H) + vi[:, None, :] + b1_ref[...][0][None, None, :],
        0.0).reshape(RB * K, H)
    h2 = jnp.maximum(
        jnp.dot(h1, w2_ref[...], preferred_element_type=jnp.float32)
        + b2_ref[...][0][None, :], 0.0)
    s = jnp.dot(h2, w3_ref[...], preferred_element_type=jnp.float32)
    s = s.reshape(RB, K) + b3_ref[0, 0]
    a = bs_ref[0, 0] * jnp.abs(s)
    m = jnp.max(a, axis=1, keepdims=True)
    ex = jnp.exp(a - m)
    scores = ex / jnp.sum(ex, axis=1, keepdims=True)
    pw = s * scores
    pw_ref[...] = pw
    pr = tg_ref[...] - eg_ref[...]
    y_ref[...] = jnp.sum(pr * pw, axis=1).reshape(1, 1, RB)
    nb = nbrs_ref[...]
    addr_ref[...] = nb[:, 0:1] * N + nb


_mlp_tc = pl.pallas_call(
    _mlp_body,
    grid=(GRID,),
    in_specs=[
        pl.BlockSpec((RB * K, D), lambda i: (i, 0)),
        pl.BlockSpec((RB, K), lambda i: (i, 0)),
        pl.BlockSpec((RB, K), lambda i: (i, 0)),
        pl.BlockSpec((RB, K), lambda i: (i, 0)),
        pl.BlockSpec((2 * D, H), lambda i: (0, 0)),
        pl.BlockSpec((1, H), lambda i: (0, 0)),
        pl.BlockSpec((H, H), lambda i: (0, 0)),
        pl.BlockSpec((1, H), lambda i: (0, 0)),
        pl.BlockSpec((H, 1), lambda i: (0, 0)),
        pl.BlockSpec((1, 1), lambda i: (0, 0)),
        pl.BlockSpec((1, 1), lambda i: (0, 0)),
    ],
    out_specs=[
        pl.BlockSpec((RB, K), lambda i: (i, 0)),
        pl.BlockSpec((1, 1, RB), lambda i: (i, 0, 0)),
        pl.BlockSpec((RB, K), lambda i: (i, 0)),
    ],
    out_shape=[
        jax.ShapeDtypeStruct((N, K), jnp.float32),
        jax.ShapeDtypeStruct((GRID, 1, RB), jnp.float32),
        jax.ShapeDtypeStruct((N, K), jnp.int32),
    ],
)


NROW = N // NW
P = 16
NPASS = NROW // P
RWORDS = P * N
CH = 128


@functools.partial(
    pl.kernel,
    out_type=jax.ShapeDtypeStruct((N * N,), jnp.float32),
    mesh=_sc_mesh,
    compiler_params=pltpu.CompilerParams(needs_layout_passes=False),
    scratch_types=[
        pltpu.VMEM((N,), jnp.int32),
        pltpu.VMEM((N + 16,), jnp.int32),
        pltpu.VMEM((N + 16,), jnp.int32),
        pltpu.VMEM((N + 16,), jnp.int32),
        pltpu.VMEM((RWORDS,), jnp.float32),
        pltpu.VMEM((2, K), jnp.int32),
        pltpu.VMEM((2, K), jnp.float32),
        pltpu.SemaphoreType.DMA,
        pltpu.SemaphoreType.DMA,
    ],
)
def _scatter_sc(cur_hbm, addr_hbm, pw_hbm, out_hbm,
                cur_v, mi_v, mc_v, pi_v, region_v, arow_v, prow_v, dsem, osem):
    wid = lax.axis_index("s") * NC + lax.axis_index("c")
    lo = wid * NROW

    pltpu.sync_copy(cur_hbm, cur_v)

    zero16f = jnp.zeros((16,), jnp.float32)
    lane = lax.iota(jnp.int32, 16)

    def _count(msk):
        return plsc.all_reduce_population_count(msk)[0]

    def _zreg_body(g, carry):
        region_v[pl.ds(g * 16, 16)] = zero16f
        return carry
    lax.fori_loop(0, RWORDS // 16, _zreg_body, 0)

    def _scan_body(g, off):
        sl = pl.ds(g * 16, 16)
        cv = cur_v[sl]
        msk = (cv >= lo) & (cv < lo + NROW)
        pc = plsc.cumsum(msk.astype(jnp.int32))
        pos = jnp.where(msk, off + pc - 1, N + lane)
        plsc.store_scatter(mi_v, [pos], lane + g * 16)
        plsc.store_scatter(mc_v, [pos], cv)
        return off + _count(msk)
    n_match = lax.fori_loop(0, N // 16, _scan_body, jnp.int32(0))

    def _pass_body(p, carry):
        lo_p = lo + p * P
        abase = lo_p * N

        def _pscan(g, off):
            sl = pl.ds(g * 16, 16)
            valid = (lane + g * 16) < n_match
            cv = mc_v[sl]
            msk = valid & (cv >= lo_p) & (cv < lo_p + P)
            pc = plsc.cumsum(msk.astype(jnp.int32))
            pos = jnp.where(msk, off + pc - 1, N + lane)
            plsc.store_scatter(pi_v, [pos], mi_v[sl])
            return off + _count(msk)
        ng = (n_match + 15) // 16
        n_pass = lax.fori_loop(0, ng, _pscan, jnp.int32(0))

        def _fire(r):
            i = pi_v[pl.ds(r, 16)][0]
            b = lax.rem(r, 2)
            da = pltpu.async_copy(addr_hbm.at[pl.ds(i * K, K)], arow_v.at[b], dsem)
            pltpu.async_copy(pw_hbm.at[pl.ds(i * K, K)], prow_v.at[b], dsem)
            return da

        def _proc(r, zero_mode):
            b = lax.rem(r, 2)
            a0 = arow_v[b, pl.ds(0, 16)] - abase
            a1 = arow_v[b, pl.ds(16, 16)] - abase
            if zero_mode:
                plsc.store_scatter(region_v, [a0], zero16f)
                plsc.store_scatter(region_v, [a1], zero16f)
            else:
                plsc.store_scatter(region_v, [a0], prow_v[b, pl.ds(0, 16)])
                plsc.store_scatter(region_v, [a1], prow_v[b, pl.ds(16, 16)])

        def _drain(r):
            i = pi_v[pl.ds(r, 16)][0]
            b = lax.rem(r, 2)
            pltpu.make_async_copy(addr_hbm.at[pl.ds(i * K, K)], arow_v.at[b], dsem).wait()
            pltpu.make_async_copy(pw_hbm.at[pl.ds(i * K, K)], prow_v.at[b], dsem).wait()

        def _run_rows(zero_mode):
            @pl.when(n_pass > 0)
            def _go():
                _fire(jnp.int32(0))

                def _rloop(r, carry2):
                    @pl.when(r + 1 < n_pass)
                    def _pf():
                        _fire(r + 1)
                    _drain(r)
                    _proc(r, zero_mode)
                    return carry2
                lax.fori_loop(0, n_pass, _rloop, 0)

        _run_rows(False)

        pltpu.async_copy(region_v, out_hbm.at[pl.ds(abase, RWORDS)], osem).wait()

        _run_rows(True)
        return carry
    lax.fori_loop(0, NPASS, _pass_body, 0)


def kernel(x, nbrs_idx, t, e_hat, W1, b1, W2, b2, W3, b3, b):
    nbrs_idx = nbrs_idx.astype(jnp.int32)
    nbrs_flat = nbrs_idx.reshape(-1)
    current = nbrs_idx[:, 0]

    z_nb, tg, eg = _gather_sc(x, nbrs_flat, t, e_hat)

    pw, y_blocks, addr = _mlp_tc(
        z_nb, nbrs_idx, tg.reshape(N, K), eg.reshape(N, K),
        W1, b1.reshape(1, H), W2, b2.reshape(1, H), W3,
        b3.reshape(1, 1), b.reshape(1, 1))

    out_flat = _scatter_sc(current, addr.reshape(E), pw.reshape(E))

    return (y_blocks.reshape(N), out_flat.reshape(N, N))


# speedup vs baseline: 8.5865x; 1.0255x over previous
import functools

import jax
import jax.numpy as jnp
from jax import lax
from jax.experimental import pallas as pl
from jax.experimental.pallas import tpu as pltpu
from jax.experimental.pallas import tpu_sc as plsc

N = 4096
D = 128
H = 128
K = 32
E = N * K

NC = 2
NS = 16
NW = NC * NS

EPW = E // NW
GJ = EPW // 128

_sc_mesh = plsc.VectorSubcoreMesh(core_axis_name="c", subcore_axis_name="s")


@functools.partial(
    pl.kernel,
    out_type=(
        jax.ShapeDtypeStruct((E, D), jnp.float32),
        jax.ShapeDtypeStruct((E,), jnp.float32),
        jax.ShapeDtypeStruct((E,), jnp.float32),
    ),
    mesh=_sc_mesh,
    compiler_params=pltpu.CompilerParams(needs_layout_passes=False),
    scratch_types=[
        pltpu.VMEM((EPW,), jnp.int32),
        pltpu.VMEM((2, 128, D), jnp.float32),
        pltpu.VMEM((EPW,), jnp.float32),
        pltpu.VMEM((EPW,), jnp.float32),
        pltpu.SemaphoreType.DMA,
        pltpu.SemaphoreType.DMA,
        pltpu.SemaphoreType.DMA,
    ],
)
def _gather_sc(x_hbm, nbrs_hbm, t_hbm, e_hbm, z_out, tg_out, eg_out,
               idx_v, ring_v, t_v, e_v, gsem, osem, tsem):
    wid = lax.axis_index("s") * NC + lax.axis_index("c")
    base = wid * EPW

    pltpu.sync_copy(nbrs_hbm.at[pl.ds(base, EPW)], idx_v)

    def _fire(j, b):
        idxref = idx_v.at[pl.ds(j * 128, 128)]
        dt = pltpu.async_copy(t_hbm.at[idxref], t_v.at[pl.ds(j * 128, 128)], tsem)
        de = pltpu.async_copy(e_hbm.at[idxref], e_v.at[pl.ds(j * 128, 128)], tsem)
        dz = pltpu.async_copy(x_hbm.at[idxref], ring_v.at[b], gsem)
        return (dz, dt, de)

    out_descs = [None, None]
    g_desc = _fire(0, 0)
    for j in range(GJ):
        b = j % 2
        nb = (j + 1) % 2
        g_cur = g_desc
        if j + 1 < GJ:
            if out_descs[nb] is not None:
                out_descs[nb].wait()
                out_descs[nb] = None
            g_desc = _fire(j + 1, nb)
        for d in g_cur:
            d.wait()
        out_descs[b] = pltpu.async_copy(
            ring_v.at[b], z_out.at[pl.ds(base + j * 128, 128)], osem)
    for od in out_descs:
        if od is not None:
            od.wait()
    pltpu.sync_copy(t_v, tg_out.at[pl.ds(base, EPW)])
    pltpu.sync_copy(e_v, eg_out.at[pl.ds(base, EPW)])


RB = 128
GRID = N // RB


def _mlp_body(z_ref, nbrs_ref, tg_ref, eg_ref, w1_ref, b1_ref, w2_ref, b2_ref,
              w3_ref, b3_ref, bs_ref, pw_ref, y_ref, addr_ref):
    z = z_ref[...]
    w1 = w1_ref[...]
    zi = z.reshape(RB, K, D)[:, 0, :]
    vi = jnp.dot(zi, w1[:D, :], preferred_element_type=jnp.float32)
    u = jnp.dot(z, w1[D:, :], preferred_element_type=jnp.float32)
    h1 = jnp.maximum(
        u.reshape(RB, K, H) + vi[:, None, :] + b1_ref[...][0][None, None, :],
        0.0).reshape(RB * K, H)
    h2 = jnp.maximum(
        jnp.dot(h1, w2_ref[...], preferred_element_type=jnp.float32)
        + b2_ref[...][0][None, :], 0.0)
    s = jnp.dot(h2, w3_ref[...], preferred_element_type=jnp.float32)
    s = s.reshape(RB, K) + b3_ref[0, 0]
    a = bs_ref[0, 0] * jnp.abs(s)
    m = jnp.max(a, axis=1, keepdims=True)
    ex = jnp.exp(a - m)
    scores = ex / jnp.sum(ex, axis=1, keepdims=True)
    pw = s * scores
    pw_ref[...] = pw
    pr = tg_ref[...] - eg_ref[...]
    y_ref[...] = jnp.sum(pr * pw, axis=1).reshape(1, 1, RB)
    nb = nbrs_ref[...]
    addr_ref[...] = nb[:, 0:1] * N + nb


_mlp_tc = pl.pallas_call(
    _mlp_body,
    grid=(GRID,),
    in_specs=[
        pl.BlockSpec((RB * K, D), lambda i: (i, 0)),
        pl.BlockSpec((RB, K), lambda i: (i, 0)),
        pl.BlockSpec((RB, K), lambda i: (i, 0)),
        pl.BlockSpec((RB, K), lambda i: (i, 0)),
        pl.BlockSpec((2 * D, H), lambda i: (0, 0)),
        pl.BlockSpec((1, H), lambda i: (0, 0)),
        pl.BlockSpec((H, H), lambda i: (0, 0)),
        pl.BlockSpec((1, H), lambda i: (0, 0)),
        pl.BlockSpec((H, 1), lambda i: (0, 0)),
        pl.BlockSpec((1, 1), lambda i: (0, 0)),
        pl.BlockSpec((1, 1), lambda i: (0, 0)),
    ],
    out_specs=[
        pl.BlockSpec((RB, K), lambda i: (i, 0)),
        pl.BlockSpec((1, 1, RB), lambda i: (i, 0, 0)),
        pl.BlockSpec((RB, K), lambda i: (i, 0)),
    ],
    out_shape=[
        jax.ShapeDtypeStruct((N, K), jnp.float32),
        jax.ShapeDtypeStruct((GRID, 1, RB), jnp.float32),
        jax.ShapeDtypeStruct((N, K), jnp.int32),
    ],
)


NROW = N // NW
P = 16
NPASS = NROW // P
RWORDS = P * N
CH = 128


@functools.partial(
    pl.kernel,
    out_type=jax.ShapeDtypeStruct((N, N), jnp.float32),
    mesh=_sc_mesh,
    compiler_params=pltpu.CompilerParams(needs_layout_passes=False),
    scratch_types=[
        pltpu.VMEM((N,), jnp.int32),
        pltpu.VMEM((N + 16,), jnp.int32),
        pltpu.VMEM((N + 16,), jnp.int32),
        pltpu.VMEM((N + 16,), jnp.int32),
        pltpu.VMEM((P, N), jnp.float32),
        pltpu.VMEM((2, K), jnp.int32),
        pltpu.VMEM((2, K), jnp.float32),
        pltpu.SemaphoreType.DMA,
        pltpu.SemaphoreType.DMA,
    ],
)
def _scatter_sc(cur_hbm, addr_hbm, pw_hbm, out_hbm,
                cur_v, mi_v, mc_v, pi_v, region_v, arow_v, prow_v, dsem, osem):
    wid = lax.axis_index("s") * NC + lax.axis_index("c")
    lo = wid * NROW

    pltpu.sync_copy(cur_hbm, cur_v)

    zero16f = jnp.zeros((16,), jnp.float32)
    lane = lax.iota(jnp.int32, 16)

    def _count(msk):
        return plsc.all_reduce_population_count(msk)[0]

    def _zero_region():
        for r in range(P):
            def _zrow(g, carry):
                region_v[r, pl.ds(g * 16, 16)] = zero16f
                return carry
            lax.fori_loop(0, N // 16, _zrow, 0)

    def _scan_body(g, off):
        sl = pl.ds(g * 16, 16)
        cv = cur_v[sl]
        msk = (cv >= lo) & (cv < lo + NROW)
        pc = plsc.cumsum(msk.astype(jnp.int32))
        pos = jnp.where(msk, off + pc - 1, N + lane)
        plsc.store_scatter(mi_v, [pos], lane + g * 16)
        plsc.store_scatter(mc_v, [pos], cv)
        return off + _count(msk)
    n_match = lax.fori_loop(0, N // 16, _scan_body, jnp.int32(0))

    def _pass_body(p, carry):
        lo_p = lo + p * P
        abase = lo_p * N
        _zero_region()

        def _pscan(g, off):
            sl = pl.ds(g * 16, 16)
            valid = (lane + g * 16) < n_match
            cv = mc_v[sl]
            msk = valid & (cv >= lo_p) & (cv < lo_p + P)
            pc = plsc.cumsum(msk.astype(jnp.int32))
            pos = jnp.where(msk, off + pc - 1, N + lane)
            plsc.store_scatter(pi_v, [pos], mi_v[sl])
            return off + _count(msk)
        ng = (n_match + 15) // 16
        n_pass = lax.fori_loop(0, ng, _pscan, jnp.int32(0))

        def _fire(r):
            i = pi_v[pl.ds(r, 16)][0]
            b = lax.rem(r, 2)
            da = pltpu.async_copy(addr_hbm.at[pl.ds(i * K, K)], arow_v.at[b], dsem)
            pltpu.async_copy(pw_hbm.at[pl.ds(i * K, K)], prow_v.at[b], dsem)
            return da

        def _proc(r):
            b = lax.rem(r, 2)
            a0 = arow_v[b, pl.ds(0, 16)]
            a1 = arow_v[b, pl.ds(16, 16)]
            plsc.store_scatter(region_v, [(a0 >> 12) - lo_p, a0 & (N - 1)],
                               prow_v[b, pl.ds(0, 16)])
            plsc.store_scatter(region_v, [(a1 >> 12) - lo_p, a1 & (N - 1)],
                               prow_v[b, pl.ds(16, 16)])

        def _drain(r):
            i = pi_v[pl.ds(r, 16)][0]
            b = lax.rem(r, 2)
            pltpu.make_async_copy(addr_hbm.at[pl.ds(i * K, K)], arow_v.at[b], dsem).wait()
            pltpu.make_async_copy(pw_hbm.at[pl.ds(i * K, K)], prow_v.at[b], dsem).wait()

        @pl.when(n_pass > 0)
        def _go():
            _fire(jnp.int32(0))

            def _rloop(r, carry2):
                @pl.when(r + 1 < n_pass)
                def _pf():
                    _fire(r + 1)
                _drain(r)
                _proc(r)
                return carry2
            lax.fori_loop(0, n_pass, _rloop, 0)

        pltpu.async_copy(region_v, out_hbm.at[pl.ds(lo_p, P), :], osem).wait()
        return carry
    lax.fori_loop(0, NPASS, _pass_body, 0)


def kernel(x, nbrs_idx, t, e_hat, W1, b1, W2, b2, W3, b3, b):
    nbrs_idx = nbrs_idx.astype(jnp.int32)
    nbrs_flat = nbrs_idx.reshape(-1)
    current = nbrs_idx[:, 0]

    z_nb, tg, eg = _gather_sc(x, nbrs_flat, t, e_hat)

    pw, y_blocks, addr = _mlp_tc(
        z_nb, nbrs_idx, tg.reshape(N, K), eg.reshape(N, K),
        W1, b1.reshape(1, H), W2, b2.reshape(1, H), W3,
        b3.reshape(1, 1), b.reshape(1, 1))

    out = _scatter_sc(current, addr.reshape(E), pw.reshape(E))

    return (y_blocks.reshape(N), out)
